# baseline (speedup 1.0000x reference)
import functools

import jax
import jax.numpy as jnp
from jax import lax
from jax.experimental import pallas as pl
from jax.experimental.pallas import tpu as pltpu
from jax.experimental.pallas import tpu_sc as plsc

B = 800000
D = 36
DP = 40
V = 324
NC = 2
NS = 16
NW = NC * NS
BPW = B // NW
CHUNK = 1000
NCH = BPW // CHUNK
GRPS = CHUNK // 16 + 1
L = 16


def _body(idx_hbm, tbl_hbm, outT_hbm, tbl_v, idx_v, outb, sem):
    wid = lax.axis_index("s") * NC + lax.axis_index("c")
    base = wid * BPW
    pltpu.sync_copy(tbl_hbm, tbl_v)

    def chunk(c, carry):
        col0 = base + c * CHUNK
        pltpu.sync_copy(idx_hbm.at[pl.ds(col0, CHUNK)], idx_v)

        def grp(g, carry2):
            off = jnp.minimum(g * L, CHUNK - L)
            addr = idx_v[pl.ds(off, L)] * DP
            for j in range(D):
                outb[j, pl.ds(off, L)] = plsc.load_gather(tbl_v, [addr + j])
            return carry2

        lax.fori_loop(0, GRPS, grp, 0)
        pltpu.sync_copy(outb, outT_hbm.at[:, pl.ds(col0, CHUNK)])
        return carry

    lax.fori_loop(0, NCH, chunk, 0)


_mesh = plsc.VectorSubcoreMesh(core_axis_name="c", subcore_axis_name="s")

_gather = functools.partial(
    pl.kernel,
    mesh=_mesh,
    out_type=jax.ShapeDtypeStruct((D, B), jnp.float32),
    scratch_types=[
        pltpu.VMEM((V * DP,), jnp.float32),
        pltpu.VMEM((CHUNK,), jnp.int32),
        pltpu.VMEM((D, CHUNK), jnp.float32),
        pltpu.SemaphoreType.DMA,
    ],
    compiler_params=pltpu.CompilerParams(
        use_tc_tiling_on_sc=False, needs_layout_passes=False
    ),
)(_body)


def kernel(atomic_numbers, gate_weight):
    tbl = jnp.pad(gate_weight, ((0, 0), (0, DP - D))).reshape(-1)
    out_t = _gather(atomic_numbers, tbl)
    return out_t.T


# speedup vs baseline: 1.0752x; 1.0752x over previous
import functools

import jax
import jax.numpy as jnp
from jax import lax
from jax.experimental import pallas as pl
from jax.experimental.pallas import tpu as pltpu
from jax.experimental.pallas import tpu_sc as plsc

B = 800000
D = 36
DP = 40
V = 324
NC = 2
NS = 16
NW = NC * NS
BPW = B // NW
CHUNK = 1008
NCH = 25
L = 16


def _body(idx_hbm, tbl_hbm, outT_hbm, tbl_v, idx_v, outb, sem):
    wid = lax.axis_index("s") * NC + lax.axis_index("c")
    base = wid * BPW
    pltpu.sync_copy(tbl_hbm, tbl_v)

    def chunk(c, carry):
        col0 = base + jnp.minimum(c * CHUNK, BPW - CHUNK)
        pltpu.sync_copy(idx_hbm.at[pl.ds(col0, CHUNK)], idx_v)

        @plsc.parallel_loop(0, CHUNK, L)
        def grp(i):
            addr = idx_v[pl.ds(i, L)] * DP
            for j in range(D):
                outb[j, pl.ds(i, L)] = plsc.load_gather(tbl_v, [addr + j])

        pltpu.sync_copy(outb, outT_hbm.at[:, pl.ds(col0, CHUNK)])
        return carry

    lax.fori_loop(0, NCH, chunk, 0)


_mesh = plsc.VectorSubcoreMesh(core_axis_name="c", subcore_axis_name="s")

_gather = functools.partial(
    pl.kernel,
    mesh=_mesh,
    out_type=jax.ShapeDtypeStruct((D, B), jnp.float32),
    scratch_types=[
        pltpu.VMEM((V * DP,), jnp.float32),
        pltpu.VMEM((CHUNK,), jnp.int32),
        pltpu.VMEM((D, CHUNK), jnp.float32),
        pltpu.SemaphoreType.DMA,
    ],
    compiler_params=pltpu.CompilerParams(
        use_tc_tiling_on_sc=False, needs_layout_passes=False
    ),
)(_body)


def kernel(atomic_numbers, gate_weight):
    tbl = jnp.pad(gate_weight, ((0, 0), (0, DP - D))).reshape(-1)
    out_t = _gather(atomic_numbers, tbl)
    return out_t.T


# speedup vs baseline: 2.9023x; 2.6993x over previous
import functools

import jax
import jax.numpy as jnp
from jax import lax
from jax.experimental import pallas as pl
from jax.experimental.pallas import tpu as pltpu
from jax.experimental.pallas import tpu_sc as plsc

B = 800000
D = 36
DP = 40
NC = 2
NS = 16
NW = NC * NS
G = 128
GPW = 196
NG = GPW * NW
BP = NG * G
NBUF = 14
NGRP = GPW // NBUF


def _body(idx_hbm, tbl_hbm, out_hbm, idx_v, rows_v, sem_g, sem_s):
    wid = lax.axis_index("s") * NC + lax.axis_index("c")
    g0 = wid * GPW
    pltpu.sync_copy(idx_hbm.at[pl.ds(g0, GPW)], idx_v)

    def group(j, c):
        jb = j * NBUF
        gathers = []
        for b in range(NBUF):
            gathers.append(
                pltpu.async_copy(
                    tbl_hbm.at[idx_v.at[jb + b]], rows_v.at[b], sem_g
                )
            )
        for b in range(NBUF):
            gathers[b].wait()
        stores = []
        for b in range(NBUF):
            stores.append(
                pltpu.async_copy(
                    rows_v.at[b],
                    out_hbm.at[pl.ds((g0 + jb + b) * G, G)],
                    sem_s,
                )
            )
        for b in range(NBUF):
            stores[b].wait()
        return c

    lax.fori_loop(0, NGRP, group, 0)


_mesh = plsc.VectorSubcoreMesh(core_axis_name="c", subcore_axis_name="s")

_gather = functools.partial(
    pl.kernel,
    mesh=_mesh,
    out_type=jax.ShapeDtypeStruct((BP, DP), jnp.float32),
    scratch_types=[
        pltpu.VMEM((GPW, G), jnp.int32),
        pltpu.VMEM((NBUF, G, DP), jnp.float32),
        pltpu.SemaphoreType.DMA,
        pltpu.SemaphoreType.DMA,
    ],
    compiler_params=pltpu.CompilerParams(use_tc_tiling_on_sc=False),
)(_body)


def kernel(atomic_numbers, gate_weight):
    tbl = jnp.pad(gate_weight, ((0, 0), (0, DP - D)))
    idx = jnp.pad(atomic_numbers, (0, BP - B)).reshape(NG, G)
    out = _gather(idx, tbl)
    return out[:B, :D]


# speedup vs baseline: 3.8717x; 1.3340x over previous
import functools

import jax
import jax.numpy as jnp
from jax import lax
from jax.experimental import pallas as pl
from jax.experimental.pallas import tpu as pltpu
from jax.experimental.pallas import tpu_sc as plsc

B = 800000
D = 36
DP = 40
NC = 2
NS = 16
NW = NC * NS
G = 128
NG = B // G
GPW = 196
NBUF = 14
NGRP = GPW // NBUF


def _body(idx_hbm, tbl_hbm, out_hbm, idx_v, rows_v, sem_g, sem_s):
    wid = lax.axis_index("s") * NC + lax.axis_index("c")
    g0 = jnp.minimum((wid * NG) // NW, NG - GPW)
    pltpu.sync_copy(idx_hbm.at[pl.ds(g0, GPW)], idx_v)

    def group(j, c):
        jb = j * NBUF
        gathers = []
        for b in range(NBUF):
            gathers.append(
                pltpu.async_copy(
                    tbl_hbm.at[idx_v.at[jb + b]], rows_v.at[b], sem_g
                )
            )
        for b in range(NBUF):
            gathers[b].wait()
        stores = []
        for b in range(NBUF):
            stores.append(
                pltpu.async_copy(
                    rows_v.at[b],
                    out_hbm.at[pl.ds((g0 + jb + b) * G, G)],
                    sem_s,
                )
            )
        for b in range(NBUF):
            stores[b].wait()
        return c

    lax.fori_loop(0, NGRP, group, 0)


_mesh = plsc.VectorSubcoreMesh(core_axis_name="c", subcore_axis_name="s")

_gather = functools.partial(
    pl.kernel,
    mesh=_mesh,
    out_type=jax.ShapeDtypeStruct((B, DP), jnp.float32),
    scratch_types=[
        pltpu.VMEM((GPW, G), jnp.int32),
        pltpu.VMEM((NBUF, G, DP), jnp.float32),
        pltpu.SemaphoreType.DMA,
        pltpu.SemaphoreType.DMA,
    ],
    compiler_params=pltpu.CompilerParams(use_tc_tiling_on_sc=False),
)(_body)


def kernel(atomic_numbers, gate_weight):
    tbl = jnp.pad(gate_weight, ((0, 0), (0, DP - D)))
    out = _gather(atomic_numbers.reshape(NG, G), tbl)
    return out[:, :D]
